# baseline (speedup 1.0000x reference)
import functools

import jax
import jax.numpy as jnp
from jax import lax
from jax.experimental import pallas as pl
from jax.experimental.pallas import tpu as pltpu
from jax.experimental.pallas import tpu_sc as plsc

_CS = 16
_NB = 5
_AHEAD = 3
_B_TC = 3
_BS = 512


def _tc_body(x_ref, w_ref, o_ref):
    o_ref[...] = x_ref[...] + w_ref[...]


def _sc_call(x, embed_weight, b_lo):
    B, S, D = x.shape
    nb = B - b_lo
    info = plsc.get_sparse_core_info()
    NC, NS, L = info.num_cores, info.num_subcores, info.num_lanes
    NW = NC * NS
    s_per_w = S // NW
    n_chunks = s_per_w // _CS
    T = n_chunks * nb

    mesh = plsc.VectorSubcoreMesh(core_axis_name="c", subcore_axis_name="s")

    @functools.partial(
        pl.kernel,
        mesh=mesh,
        out_type=jax.ShapeDtypeStruct((nb, S, D), jnp.float32),
        scratch_types=(
            [pltpu.VMEM((_CS, D), jnp.float32) for _ in range(2 + _NB)]
            + [pltpu.SemaphoreType.DMA for _ in range(2 + 2 * _NB)]
        ),
    )
    def k(x_hbm, w_hbm, out_hbm, *bufs_and_sems):
        wbufs = list(bufs_and_sems[0:2])
        xbufs = list(bufs_and_sems[2:2 + _NB])
        sems = bufs_and_sems[2 + _NB:]
        wsems = list(sems[0:2])
        xisems = list(sems[2:2 + _NB])
        xosems = list(sems[2 + _NB:2 + 2 * _NB])

        wid = lax.axis_index("s") * NC + lax.axis_index("c")
        s0 = wid * s_per_w

        def s_lo(c):
            return s0 + c * _CS

        d_shift = D.bit_length() - 1

        def add_tile(xb, wb):
            @plsc.parallel_loop(0, _CS * D, step=L, unroll=8)
            def _(i):
                r = i >> d_shift
                o = pl.multiple_of(i & (D - 1), L)
                plsc.addupdate(xb.at[r, pl.ds(o, L)], wb[r, pl.ds(o, L)])

        def start_in(t):
            c, b = divmod(t, nb)
            return pltpu.async_copy(
                x_hbm.at[b_lo + b, pl.ds(s_lo(c), _CS)], xbufs[t % _NB],
                xisems[t % _NB])

        w_h = [None, None]
        xi_h = [None] * _NB
        xo_h = [None] * _NB

        w_h[0] = pltpu.async_copy(
            w_hbm.at[pl.ds(s_lo(0), _CS)], wbufs[0], wsems[0])
        for t in range(min(_AHEAD, T)):
            xi_h[t % _NB] = start_in(t)

        for t in range(T):
            p = t % _NB
            c, b = divmod(t, nb)
            if t + _AHEAD < T:
                q = (t + _AHEAD) % _NB
                if xo_h[q] is not None:
                    xo_h[q].wait()
                    xo_h[q] = None
                xi_h[q] = start_in(t + _AHEAD)
            if b == 0:
                w_h[c % 2].wait()
                if c + 1 < n_chunks:
                    w_h[(c + 1) % 2] = pltpu.async_copy(
                        w_hbm.at[pl.ds(s_lo(c + 1), _CS)],
                        wbufs[(c + 1) % 2], wsems[(c + 1) % 2])
            xi_h[p].wait()
            add_tile(xbufs[p], wbufs[c % 2])
            xo_h[p] = pltpu.async_copy(
                xbufs[p], out_hbm.at[b, pl.ds(s_lo(c), _CS)], xosems[p])

        for p in range(_NB):
            if xo_h[p] is not None:
                xo_h[p].wait()

    return k(x, embed_weight)


def _tc_call(x, embed_weight, b_hi):
    B, S, D = x.shape
    grid = (S // _BS, b_hi)
    return pl.pallas_call(
        _tc_body,
        grid=grid,
        in_specs=[
            pl.BlockSpec((1, _BS, D), lambda s, b: (b, s, 0)),
            pl.BlockSpec((_BS, D), lambda s, b: (s, 0)),
        ],
        out_specs=pl.BlockSpec((1, _BS, D), lambda s, b: (b, s, 0)),
        out_shape=jax.ShapeDtypeStruct((b_hi, S, D), x.dtype),
    )(x, embed_weight)


def kernel(x, embed_weight):
    sc_out = _sc_call(x, embed_weight, _B_TC)
    tc_out = _tc_call(x, embed_weight, _B_TC)
    return jnp.concatenate([tc_out, sc_out], axis=0)


# speedup vs baseline: 2.1717x; 2.1717x over previous
import jax
import jax.numpy as jnp
from jax.experimental import pallas as pl


_BS = 512


def kernel(x, embed_weight):
    B, S, D = x.shape

    def body(x_ref, w_ref, o_ref):
        s = pl.program_id(0)
        o_ref[...] = x_ref[...] + w_ref[pl.ds(s * _BS, _BS), :][None]

    grid = (S // _BS, B)
    return pl.pallas_call(
        body,
        grid=grid,
        in_specs=[
            pl.BlockSpec((1, _BS, D), lambda s, b: (b, s, 0)),
            pl.BlockSpec((S, D), lambda s, b: (0, 0)),
        ],
        out_specs=pl.BlockSpec((1, _BS, D), lambda s, b: (b, s, 0)),
        out_shape=jax.ShapeDtypeStruct((B, S, D), x.dtype),
    )(x, embed_weight)


# speedup vs baseline: 2.3505x; 1.0823x over previous
import jax
import jax.numpy as jnp
from jax.experimental import pallas as pl


_BS = 1024


def kernel(x, embed_weight):
    B, S, D = x.shape

    def body(x_ref, w_ref, o_ref):
        s = pl.program_id(0)
        o_ref[...] = x_ref[...] + w_ref[pl.ds(s * _BS, _BS), :][None]

    grid = (S // _BS, B)
    return pl.pallas_call(
        body,
        grid=grid,
        in_specs=[
            pl.BlockSpec((1, _BS, D), lambda s, b: (b, s, 0)),
            pl.BlockSpec((S, D), lambda s, b: (0, 0)),
        ],
        out_specs=pl.BlockSpec((1, _BS, D), lambda s, b: (b, s, 0)),
        out_shape=jax.ShapeDtypeStruct((B, S, D), x.dtype),
    )(x, embed_weight)


# speedup vs baseline: 2.4494x; 1.0421x over previous
import jax
import jax.numpy as jnp
from jax.experimental import pallas as pl


_BS = 2048


def kernel(x, embed_weight):
    B, S, D = x.shape

    def body(x_ref, w_ref, o_ref):
        s = pl.program_id(0)
        o_ref[...] = x_ref[...] + w_ref[pl.ds(s * _BS, _BS), :][None]

    grid = (S // _BS, B)
    return pl.pallas_call(
        body,
        grid=grid,
        in_specs=[
            pl.BlockSpec((1, _BS, D), lambda s, b: (b, s, 0)),
            pl.BlockSpec((S, D), lambda s, b: (0, 0)),
        ],
        out_specs=pl.BlockSpec((1, _BS, D), lambda s, b: (b, s, 0)),
        out_shape=jax.ShapeDtypeStruct((B, S, D), x.dtype),
    )(x, embed_weight)
